# baseline (speedup 1.0000x reference)
import jax
import jax.numpy as jnp
import numpy as np
from jax import lax
from jax.experimental import pallas as pl
from jax.experimental.pallas import tpu as pltpu
from jax.experimental.pallas import tpu_sc as plsc

_PERM64 = np.array([
    35, 45, 31, 63, 7, 4, 29, 44, 16, 58, 37, 19, 61, 2, 34, 5,
    30, 42, 3, 39, 56, 22, 6, 54, 18, 10, 11, 53, 32, 15, 49, 50,
    20, 43, 8, 24, 9, 40, 59, 25, 13, 52, 62, 60, 47, 33, 14, 17,
    38, 23, 0, 41, 21, 26, 57, 1, 28, 48, 36, 55, 51, 27, 12, 46,
], dtype=np.int32)

_NC = 2
_NS = 16
_NW = _NC * _NS
_L = 16

_N = 100000
_B = 65536
_K = 64
_S = 32
_CHUNK = 4096
_NCHUNK = _B // _CHUNK


def _sc_body(adjt_ref, ids_ref, perm_ref, ns_ref, out_ref,
             col_v, perm_v, ns_v, ids_a, ids_b, out_a, out_b, ids_sh,
             csem, gsem_a, gsem_b, ssem_a, ssem_b, shsem):
    sid = lax.axis_index("s")
    wid = sid * _NC + lax.axis_index("c")
    pltpu.sync_copy(perm_ref, perm_v)
    pltpu.sync_copy(ns_ref, ns_v)
    lane = lax.iota(jnp.int32, _L)
    ns_b = plsc.load_gather(ns_v, [jnp.zeros((_L,), jnp.int32)])
    ns = jnp.sum(jnp.where(lane == 0, ns_b, 0))
    jidx = jnp.minimum(wid, ns - 1)
    grp = (jidx // _L) * _L
    pv = perm_v[pl.ds(grp, _L)]
    c = jnp.sum(jnp.where(lane == jidx - grp, pv, 0))

    col_dma = pltpu.async_copy(adjt_ref.at[c], col_v, csem)

    @pl.when(sid == 0)
    def _stage_ids():
        pltpu.async_copy(ids_ref, ids_sh, shsem).wait()

    plsc.subcore_barrier()

    idbufs = (ids_a, ids_b)
    obufs = (out_a, out_b)
    gsems = (gsem_a, gsem_b)
    ssems = (ssem_a, ssem_b)

    def ids_copy(c, b):
        return pltpu.make_async_copy(
            ids_sh.at[pl.ds(c * _CHUNK, _CHUNK)], idbufs[b], gsems[b])

    def out_copy(c, b):
        return pltpu.make_async_copy(
            obufs[b], out_ref.at[wid, pl.ds(c * _CHUNK, _CHUNK)], ssems[b])

    ids_copy(0, 0).start()
    ids_copy(1, 1).start()
    col_dma.wait()

    def chunk_body(kk, carry):
        for b in range(2):
            c = kk * 2 + b
            ids_copy(c, b).wait()

            @pl.when(kk > 0)
            def _wait_out():
                out_copy(c, b).wait()

            idv = idbufs[b]
            ob = obufs[b]

            @plsc.parallel_loop(0, _CHUNK, step=_L, unroll=8)
            def grp_body(i, idv=idv, ob=ob):
                idx = idv[pl.ds(i, _L)]
                ob[pl.ds(i, _L)] = plsc.load_gather(col_v, [idx])

            out_copy(c, b).start()

            @pl.when(c + 2 < _NCHUNK)
            def _next_ids():
                ids_copy(c + 2, b).start()
        return carry

    lax.fori_loop(0, _NCHUNK // 2, chunk_body, 0)
    out_copy(_NCHUNK - 2, 0).wait()
    out_copy(_NCHUNK - 1, 1).wait()


def _sc_gather(adjt, ids, perm, ns1):
    mesh = plsc.VectorSubcoreMesh(
        core_axis_name="c", subcore_axis_name="s",
        num_cores=_NC, num_subcores=_NS)
    i32 = jnp.int32
    return pl.kernel(
        _sc_body,
        out_type=jax.ShapeDtypeStruct((_S, _B), i32),
        mesh=mesh,
        compiler_params=pltpu.CompilerParams(
            needs_layout_passes=False, use_tc_tiling_on_sc=True),
        scratch_types=[
            pltpu.VMEM((_N,), i32),
            pltpu.VMEM((_K,), i32),
            pltpu.VMEM((1,), i32),
            pltpu.VMEM((_CHUNK,), i32),
            pltpu.VMEM((_CHUNK,), i32),
            pltpu.VMEM((_CHUNK,), i32),
            pltpu.VMEM((_CHUNK,), i32),
            pltpu.VMEM_SHARED((_B,), i32),
            pltpu.SemaphoreType.DMA,
            pltpu.SemaphoreType.DMA,
            pltpu.SemaphoreType.DMA,
            pltpu.SemaphoreType.DMA,
            pltpu.SemaphoreType.DMA,
            pltpu.SemaphoreType.DMA,
        ],
    )(adjt, ids, perm, ns1)


def kernel(ids, num_samples, adj_info):
    perm = jnp.asarray(_PERM64, jnp.int32)
    ns1 = jnp.reshape(num_samples, (1,)).astype(jnp.int32)
    out_t = _sc_gather(adj_info.T.astype(jnp.int32), ids.astype(jnp.int32),
                       perm, ns1)
    return out_t.T.astype(adj_info.dtype)


# speedup vs baseline: 1.0297x; 1.0297x over previous
import jax
import jax.numpy as jnp
import numpy as np
from jax import lax
from jax.experimental import pallas as pl
from jax.experimental.pallas import tpu as pltpu
from jax.experimental.pallas import tpu_sc as plsc

_PERM64 = np.array([
    35, 45, 31, 63, 7, 4, 29, 44, 16, 58, 37, 19, 61, 2, 34, 5,
    30, 42, 3, 39, 56, 22, 6, 54, 18, 10, 11, 53, 32, 15, 49, 50,
    20, 43, 8, 24, 9, 40, 59, 25, 13, 52, 62, 60, 47, 33, 14, 17,
    38, 23, 0, 41, 21, 26, 57, 1, 28, 48, 36, 55, 51, 27, 12, 46,
], dtype=np.int32)

_NC = 2
_NS = 16
_NW = _NC * _NS
_L = 16

_N = 100000
_B = 65536
_K = 64
_S = 32
_CHUNK = 2048
_NCHUNK = _B // _CHUNK


def _sc_body(adjt_ref, ids_ref, pos_ref, out_ref,
             col_v, pos_v, ids_a, ids_b, out_a, out_b, ids_sh,
             csem, gsem_a, gsem_b, ssem_a, ssem_b, shsem):
    sid = lax.axis_index("s")
    wid = sid * _NC + lax.axis_index("c")
    pltpu.sync_copy(pos_ref, pos_v)
    grp = (wid // _L) * _L
    pv = pos_v[pl.ds(grp, _L)]
    lane = lax.iota(jnp.int32, _L)
    c = jnp.sum(jnp.where(lane == wid - grp, pv, 0))

    col_dma = pltpu.async_copy(adjt_ref.at[c], col_v, csem)

    @pl.when(sid == 0)
    def _stage_ids():
        pltpu.async_copy(ids_ref, ids_sh, shsem).wait()

    plsc.subcore_barrier()

    idbufs = (ids_a, ids_b)
    obufs = (out_a, out_b)
    gsems = (gsem_a, gsem_b)
    ssems = (ssem_a, ssem_b)

    def ids_copy(c, b):
        return pltpu.make_async_copy(
            ids_sh.at[pl.ds(c * _CHUNK, _CHUNK)], idbufs[b], gsems[b])

    def out_copy(c, b):
        return pltpu.make_async_copy(
            obufs[b], out_ref.at[wid, pl.ds(c * _CHUNK, _CHUNK)], ssems[b])

    ids_copy(0, 0).start()
    ids_copy(1, 1).start()
    col_dma.wait()

    def chunk_body(kk, carry):
        for b in range(2):
            c = kk * 2 + b
            ids_copy(c, b).wait()

            @pl.when(kk > 0)
            def _wait_out():
                out_copy(c, b).wait()

            idv = idbufs[b]
            ob = obufs[b]

            @plsc.parallel_loop(0, _CHUNK, step=_L, unroll=8)
            def grp_body(i, idv=idv, ob=ob):
                idx = idv[pl.ds(i, _L)]
                ob[pl.ds(i, _L)] = plsc.load_gather(col_v, [idx])

            out_copy(c, b).start()

            @pl.when(c + 2 < _NCHUNK)
            def _next_ids():
                ids_copy(c + 2, b).start()
        return carry

    lax.fori_loop(0, _NCHUNK // 2, chunk_body, 0)
    out_copy(_NCHUNK - 2, 0).wait()
    out_copy(_NCHUNK - 1, 1).wait()


def _sc_gather(adjt, ids, pos):
    mesh = plsc.VectorSubcoreMesh(
        core_axis_name="c", subcore_axis_name="s",
        num_cores=_NC, num_subcores=_NS)
    i32 = jnp.int32
    return pl.kernel(
        _sc_body,
        out_type=jax.ShapeDtypeStruct((_S, _B), i32),
        mesh=mesh,
        compiler_params=pltpu.CompilerParams(
            needs_layout_passes=False, use_tc_tiling_on_sc=True),
        scratch_types=[
            pltpu.VMEM((_N,), i32),
            pltpu.VMEM((_S,), i32),
            pltpu.VMEM((_CHUNK,), i32),
            pltpu.VMEM((_CHUNK,), i32),
            pltpu.VMEM((_CHUNK,), i32),
            pltpu.VMEM((_CHUNK,), i32),
            pltpu.VMEM_SHARED((_B,), i32),
            pltpu.SemaphoreType.DMA,
            pltpu.SemaphoreType.DMA,
            pltpu.SemaphoreType.DMA,
            pltpu.SemaphoreType.DMA,
            pltpu.SemaphoreType.DMA,
            pltpu.SemaphoreType.DMA,
        ],
    )(adjt, ids, pos)


def kernel(ids, num_samples, adj_info):
    perm = jnp.asarray(_PERM64, jnp.int32)
    cols = perm[jnp.minimum(jnp.arange(_S), num_samples - 1)].astype(jnp.int32)
    out_t = _sc_gather(adj_info.T.astype(jnp.int32), ids.astype(jnp.int32),
                       cols)
    return out_t.T.astype(adj_info.dtype)
